# Initial kernel scaffold; baseline (speedup 1.0000x reference)
import jax
import jax.numpy as jnp
from jax.experimental import pallas as pl


def kernel(x, edge_index, W_in, b_in, W_msg, b_msg, W_upd, b_upd):
    raise NotImplementedError("write your pallas kernel here")


# speedup vs baseline: 3.6002x; 3.6002x over previous
import functools

import jax
import jax.numpy as jnp
from jax import lax
from jax.experimental import pallas as pl
from jax.experimental.pallas import tpu as pltpu
from jax.experimental.pallas import tpu_sc as plsc

_NC = 2
_NS = 16
_NW = _NC * _NS
_CHUNK = 128


def _tc_init(x, W_in, b_in, W_msg0, b_msg0, block_rows):
    n, d = x.shape
    ds = W_in.shape[1]

    def body(x_ref, wi_ref, bi_ref, wm_ref, bm_ref, state_ref, msg_ref):
        s = jnp.maximum(
            jnp.dot(x_ref[...], wi_ref[...], preferred_element_type=jnp.float32)
            + bi_ref[...], 0.0)
        state_ref[...] = s
        msg_ref[...] = jnp.maximum(
            jnp.dot(s, wm_ref[...], preferred_element_type=jnp.float32)
            + bm_ref[...], 0.0)

    grid = (n // block_rows,)
    return pl.pallas_call(
        body,
        grid=grid,
        in_specs=[
            pl.BlockSpec((block_rows, d), lambda i: (i, 0)),
            pl.BlockSpec((d, ds), lambda i: (0, 0)),
            pl.BlockSpec((1, ds), lambda i: (0, 0)),
            pl.BlockSpec((ds, ds), lambda i: (0, 0)),
            pl.BlockSpec((1, ds), lambda i: (0, 0)),
        ],
        out_specs=[
            pl.BlockSpec((block_rows, ds), lambda i: (i, 0)),
            pl.BlockSpec((block_rows, ds), lambda i: (i, 0)),
        ],
        out_shape=[
            jax.ShapeDtypeStruct((n, ds), jnp.float32),
            jax.ShapeDtypeStruct((n, ds), jnp.float32),
        ],
    )(x, W_in, b_in.reshape(1, ds), W_msg0, b_msg0.reshape(1, ds))


def _tc_update(agg2, state, W_upd, b_upd, W_msg_next, b_msg_next, block_rows):
    n, ds = state.shape
    with_msg = W_msg_next is not None

    def body(agg_ref, st_ref, wu_ref, bu_ref, *rest):
        if with_msg:
            wm_ref, bm_ref, state_ref, msg_ref = rest
        else:
            (state_ref,) = rest
        agg = agg_ref[0] + agg_ref[1]
        upd = jnp.maximum(
            jnp.dot(agg, wu_ref[...], preferred_element_type=jnp.float32)
            + bu_ref[...], 0.0)
        s = st_ref[...] + upd
        state_ref[...] = s
        if with_msg:
            msg_ref[...] = jnp.maximum(
                jnp.dot(s, rest[0][...], preferred_element_type=jnp.float32)
                + rest[1][...], 0.0)

    grid = (n // block_rows,)
    in_specs = [
        pl.BlockSpec((2, block_rows, ds), lambda i: (0, i, 0)),
        pl.BlockSpec((block_rows, ds), lambda i: (i, 0)),
        pl.BlockSpec((ds, ds), lambda i: (0, 0)),
        pl.BlockSpec((1, ds), lambda i: (0, 0)),
    ]
    out_specs = [pl.BlockSpec((block_rows, ds), lambda i: (i, 0))]
    out_shape = [jax.ShapeDtypeStruct((n, ds), jnp.float32)]
    args = [agg2, state, W_upd, b_upd.reshape(1, ds)]
    if with_msg:
        in_specs += [
            pl.BlockSpec((ds, ds), lambda i: (0, 0)),
            pl.BlockSpec((1, ds), lambda i: (0, 0)),
        ]
        out_specs += [pl.BlockSpec((block_rows, ds), lambda i: (i, 0))]
        out_shape += [jax.ShapeDtypeStruct((n, ds), jnp.float32)]
        args += [W_msg_next, b_msg_next.reshape(1, ds)]
    res = pl.pallas_call(
        body, grid=grid, in_specs=in_specs, out_specs=out_specs,
        out_shape=out_shape)(*args)
    return res if with_msg else (res[0], None)


def _make_sc_scatter(n_nodes, d_state, e_pad, n_pad_rows):
    epw = e_pad // _NW
    nchunk = epw // _CHUNK
    assert nchunk % 2 == 0
    npairs = nchunk // 2
    tile_rows = (n_nodes // _NS) & ~7
    rem_rows = n_nodes - tile_rows * _NS
    rem_off = tile_rows * _NS
    mesh = plsc.VectorSubcoreMesh(core_axis_name="c", subcore_axis_name="s")

    @functools.partial(
        pl.kernel,
        out_type=jax.ShapeDtypeStruct((_NC, n_nodes, d_state), jnp.float32),
        mesh=mesh,
        scratch_types=[
            pltpu.VMEM_SHARED((n_pad_rows, d_state), jnp.float32),
            pltpu.VMEM((2, _CHUNK), jnp.int32),
            pltpu.VMEM((2, _CHUNK), jnp.int32),
            pltpu.VMEM((2, _CHUNK, d_state), jnp.float32),
            pltpu.SemaphoreType.DMA((2,)),
        ],
    )
    def sc_scatter(msg_hbm, src_hbm, dst_hbm, zeros_hbm, out_hbm,
                   agg_sh, src_v, dst_v, rows_v, gsem):
        c = lax.axis_index("c")
        s = lax.axis_index("s")
        wid = s * _NC + c
        pltpu.sync_copy(zeros_hbm.at[pl.ds(s * tile_rows, tile_rows)],
                        agg_sh.at[pl.ds(s * tile_rows, tile_rows)])
        if rem_rows:
            @pl.when(s == 0)
            def _():
                pltpu.sync_copy(zeros_hbm.at[pl.ds(rem_off, rem_rows)],
                                agg_sh.at[pl.ds(rem_off, rem_rows)])
        plsc.subcore_barrier()
        base = wid * epw

        def start_gather(b, chunk):
            off = base + chunk * _CHUNK
            pltpu.sync_copy(src_hbm.at[pl.ds(off, _CHUNK)], src_v.at[b])
            pltpu.sync_copy(dst_hbm.at[pl.ds(off, _CHUNK)], dst_v.at[b])
            pltpu.async_copy(msg_hbm.at[src_v.at[b]], rows_v.at[b],
                             gsem.at[b])

        def finish_chunk(b):
            pltpu.make_async_copy(msg_hbm.at[src_v.at[b]], rows_v.at[b],
                                  gsem.at[b]).wait()
            pltpu.sync_copy(rows_v.at[b], agg_sh.at[dst_v.at[b]], add=True)

        for b in range(2):
            start_gather(b, b)

        def body(i, carry):
            for b in range(2):
                finish_chunk(b)
                start_gather(b, 2 * i + b + 2)
            return carry

        lax.fori_loop(0, npairs - 1, body, 0)
        for b in range(2):
            finish_chunk(b)
        plsc.subcore_barrier()
        pltpu.sync_copy(agg_sh.at[pl.ds(s * tile_rows, tile_rows)],
                        out_hbm.at[c, pl.ds(s * tile_rows, tile_rows)])
        if rem_rows:
            @pl.when(s == 0)
            def _():
                pltpu.sync_copy(agg_sh.at[pl.ds(rem_off, rem_rows)],
                                out_hbm.at[c, pl.ds(rem_off, rem_rows)])

    return sc_scatter


def kernel(x, edge_index, W_in, b_in, W_msg, b_msg, W_upd, b_upd):
    n_nodes, d_feat = x.shape
    d_state = W_in.shape[1]
    n_rounds = W_msg.shape[0]
    n_edges = edge_index.shape[1]

    unit = _NW * _CHUNK * 2
    e_pad = ((n_edges + unit - 1) // unit) * unit
    pad = e_pad - n_edges
    src = jnp.concatenate(
        [edge_index[0], jnp.zeros((pad,), jnp.int32)]) if pad else edge_index[0]
    dst = jnp.concatenate(
        [edge_index[1], jnp.full((pad,), n_nodes, jnp.int32)]) if pad else edge_index[1]

    n_pad_rows = n_nodes + 8
    zeros_hbm = jnp.zeros((n_nodes, d_state), jnp.float32)

    sc_scatter = _make_sc_scatter(n_nodes, d_state, e_pad, n_pad_rows)

    block_rows = 1000
    state, msg = _tc_init(x, W_in, b_in, W_msg[0], b_msg[0], block_rows)
    for r in range(n_rounds):
        agg2 = sc_scatter(msg, src, dst, zeros_hbm)
        if r + 1 < n_rounds:
            state, msg = _tc_update(agg2, state, W_upd[r], b_upd[r],
                                    W_msg[r + 1], b_msg[r + 1], block_rows)
        else:
            state, _ = _tc_update(agg2, state, W_upd[r], b_upd[r],
                                  None, None, block_rows)
    return state


# speedup vs baseline: 3.6004x; 1.0001x over previous
import functools

import jax
import jax.numpy as jnp
from jax import lax
from jax.experimental import pallas as pl
from jax.experimental.pallas import tpu as pltpu
from jax.experimental.pallas import tpu_sc as plsc

_NC = 2
_NS = 16
_NW = _NC * _NS
_CHUNK = 128


def _tc_init(x, W_in, b_in, W_msg0, b_msg0, block_rows):
    n, d = x.shape
    ds = W_in.shape[1]

    def body(x_ref, wi_ref, bi_ref, wm_ref, bm_ref, state_ref, msg_ref):
        s = jnp.maximum(
            jnp.dot(x_ref[...], wi_ref[...], preferred_element_type=jnp.float32)
            + bi_ref[...], 0.0)
        state_ref[...] = s
        msg_ref[...] = jnp.maximum(
            jnp.dot(s, wm_ref[...], preferred_element_type=jnp.float32)
            + bm_ref[...], 0.0)

    grid = (n // block_rows,)
    return pl.pallas_call(
        body,
        grid=grid,
        in_specs=[
            pl.BlockSpec((block_rows, d), lambda i: (i, 0)),
            pl.BlockSpec((d, ds), lambda i: (0, 0)),
            pl.BlockSpec((1, ds), lambda i: (0, 0)),
            pl.BlockSpec((ds, ds), lambda i: (0, 0)),
            pl.BlockSpec((1, ds), lambda i: (0, 0)),
        ],
        out_specs=[
            pl.BlockSpec((block_rows, ds), lambda i: (i, 0)),
            pl.BlockSpec((block_rows, ds), lambda i: (i, 0)),
        ],
        out_shape=[
            jax.ShapeDtypeStruct((n, ds), jnp.float32),
            jax.ShapeDtypeStruct((n, ds), jnp.float32),
        ],
    )(x, W_in, b_in.reshape(1, ds), W_msg0, b_msg0.reshape(1, ds))


def _tc_update(agg2, state, W_upd, b_upd, W_msg_next, b_msg_next, block_rows):
    n, ds = state.shape
    with_msg = W_msg_next is not None

    def body(agg_ref, st_ref, wu_ref, bu_ref, *rest):
        if with_msg:
            wm_ref, bm_ref, state_ref, msg_ref = rest
        else:
            (state_ref,) = rest
        agg = agg_ref[0] + agg_ref[1]
        upd = jnp.maximum(
            jnp.dot(agg, wu_ref[...], preferred_element_type=jnp.float32)
            + bu_ref[...], 0.0)
        s = st_ref[...] + upd
        state_ref[...] = s
        if with_msg:
            msg_ref[...] = jnp.maximum(
                jnp.dot(s, rest[0][...], preferred_element_type=jnp.float32)
                + rest[1][...], 0.0)

    grid = (n // block_rows,)
    in_specs = [
        pl.BlockSpec((2, block_rows, ds), lambda i: (0, i, 0)),
        pl.BlockSpec((block_rows, ds), lambda i: (i, 0)),
        pl.BlockSpec((ds, ds), lambda i: (0, 0)),
        pl.BlockSpec((1, ds), lambda i: (0, 0)),
    ]
    out_specs = [pl.BlockSpec((block_rows, ds), lambda i: (i, 0))]
    out_shape = [jax.ShapeDtypeStruct((n, ds), jnp.float32)]
    args = [agg2, state, W_upd, b_upd.reshape(1, ds)]
    if with_msg:
        in_specs += [
            pl.BlockSpec((ds, ds), lambda i: (0, 0)),
            pl.BlockSpec((1, ds), lambda i: (0, 0)),
        ]
        out_specs += [pl.BlockSpec((block_rows, ds), lambda i: (i, 0))]
        out_shape += [jax.ShapeDtypeStruct((n, ds), jnp.float32)]
        args += [W_msg_next, b_msg_next.reshape(1, ds)]
    res = pl.pallas_call(
        body, grid=grid, in_specs=in_specs, out_specs=out_specs,
        out_shape=out_shape)(*args)
    return res if with_msg else (res[0], None)


def _make_sc_scatter(n_nodes, d_state, e_pad, n_pad_rows):
    epw = e_pad // _NW
    nchunk = epw // _CHUNK
    assert nchunk % 2 == 0
    npairs = nchunk // 2
    tile_rows = (n_nodes // _NS) & ~7
    rem_rows = n_nodes - tile_rows * _NS
    rem_off = tile_rows * _NS
    mesh = plsc.VectorSubcoreMesh(core_axis_name="c", subcore_axis_name="s")

    @functools.partial(
        pl.kernel,
        out_type=jax.ShapeDtypeStruct((_NC, n_nodes, d_state), jnp.float32),
        mesh=mesh,
        scratch_types=[
            pltpu.VMEM_SHARED((n_pad_rows, d_state), jnp.float32),
            pltpu.VMEM((2, _CHUNK), jnp.int32),
            pltpu.VMEM((2, _CHUNK), jnp.int32),
            pltpu.VMEM((2, _CHUNK, d_state), jnp.float32),
            pltpu.SemaphoreType.DMA((2,)),
        ],
    )
    def sc_scatter(msg_hbm, src_hbm, dst_hbm, zeros_hbm, out_hbm,
                   agg_sh, src_v, dst_v, rows_v, gsem):
        c = lax.axis_index("c")
        s = lax.axis_index("s")
        wid = s * _NC + c
        pltpu.sync_copy(zeros_hbm.at[pl.ds(s * tile_rows, tile_rows)],
                        agg_sh.at[pl.ds(s * tile_rows, tile_rows)])
        if rem_rows:
            @pl.when(s == 0)
            def _():
                pltpu.sync_copy(zeros_hbm.at[pl.ds(rem_off, rem_rows)],
                                agg_sh.at[pl.ds(rem_off, rem_rows)])
        plsc.subcore_barrier()
        base = wid * epw

        def start_gather(b, chunk):
            off = base + chunk * _CHUNK
            pltpu.sync_copy(src_hbm.at[pl.ds(off, _CHUNK)], src_v.at[b])
            pltpu.sync_copy(dst_hbm.at[pl.ds(off, _CHUNK)], dst_v.at[b])
            pltpu.async_copy(msg_hbm.at[src_v.at[b]], rows_v.at[b],
                             gsem.at[b])

        def finish_chunk(b):
            pltpu.make_async_copy(msg_hbm.at[src_v.at[b]], rows_v.at[b],
                                  gsem.at[b]).wait()
            pltpu.sync_copy(rows_v.at[b], agg_sh.at[dst_v.at[b]], add=True)

        for b in range(2):
            start_gather(b, b)

        def body(i, carry):
            for b in range(2):
                finish_chunk(b)
                start_gather(b, 2 * i + b + 2)
            return carry

        lax.fori_loop(0, npairs - 1, body, 0)
        for b in range(2):
            finish_chunk(b)
        plsc.subcore_barrier()
        pltpu.sync_copy(agg_sh.at[pl.ds(s * tile_rows, tile_rows)],
                        out_hbm.at[c, pl.ds(s * tile_rows, tile_rows)])
        if rem_rows:
            @pl.when(s == 0)
            def _():
                pltpu.sync_copy(agg_sh.at[pl.ds(rem_off, rem_rows)],
                                out_hbm.at[c, pl.ds(rem_off, rem_rows)])

    return sc_scatter


def kernel(x, edge_index, W_in, b_in, W_msg, b_msg, W_upd, b_upd):
    n_nodes, d_feat = x.shape
    d_state = W_in.shape[1]
    n_rounds = W_msg.shape[0]
    n_edges = edge_index.shape[1]

    unit = _NW * _CHUNK * 2
    e_pad = ((n_edges + unit - 1) // unit) * unit
    pad = e_pad - n_edges
    junk_rows = 1024
    src = jnp.concatenate(
        [edge_index[0], jnp.zeros((pad,), jnp.int32)]) if pad else edge_index[0]
    dst = jnp.concatenate(
        [edge_index[1],
         n_nodes + (jnp.arange(pad, dtype=jnp.int32) % junk_rows)]
    ) if pad else edge_index[1]

    n_pad_rows = n_nodes + junk_rows
    zeros_hbm = jnp.zeros((n_nodes, d_state), jnp.float32)

    sc_scatter = _make_sc_scatter(n_nodes, d_state, e_pad, n_pad_rows)

    block_rows = 1000
    state, msg = _tc_init(x, W_in, b_in, W_msg[0], b_msg[0], block_rows)
    for r in range(n_rounds):
        agg2 = sc_scatter(msg, src, dst, zeros_hbm)
        if r + 1 < n_rounds:
            state, msg = _tc_update(agg2, state, W_upd[r], b_upd[r],
                                    W_msg[r + 1], b_msg[r + 1], block_rows)
        else:
            state, _ = _tc_update(agg2, state, W_upd[r], b_upd[r],
                                  None, None, block_rows)
    return state
